# Initial kernel scaffold; baseline (speedup 1.0000x reference)
import jax
import jax.numpy as jnp
from jax.experimental import pallas as pl


def kernel(target_triples, pos_context, neg_context, W_target_head, W_target_tail, W_target_rel, W_context_head, W_context_tail, W_context_rel):
    raise NotImplementedError("write your pallas kernel here")


# speedup vs baseline: 1.0841x; 1.0841x over previous
import functools

import jax
import jax.numpy as jnp
from jax import lax
from jax.experimental import pallas as pl
from jax.experimental.pallas import tpu as pltpu
from jax.experimental.pallas import tpu_sc as plsc

_EPS = 1e-15

_NC, _NS, _L = 2, 16, 16
_NW = _NC * _NS

_B = 4096
_C = 50
_D = 64
_BPW = _B // _NW
_SEGS_PER_CHUNK = 2
_ROWS = _SEGS_PER_CHUNK * _C
_CHUNKS = _BPW // _SEGS_PER_CHUNK


def _sc_body(Wt0, Wt1, Wt2, Wc0, Wc1, Wc2, ctx_idx, tgt_idx, out,
             idx_v, rows_v, acc_v, tidx_v, trows_v, sem):
    cid = lax.axis_index("c")
    sid = lax.axis_index("s")
    wid = sid * _NC + cid
    base = wid * _BPW

    tgt_tabs = (Wt0, Wt1, Wt2)
    ctx_tabs = (Wc0, Wc1, Wc2)

    for comp in range(3):
        pltpu.sync_copy(tgt_idx.at[comp, wid], tidx_v)
        pltpu.async_copy(tgt_tabs[comp].at[tidx_v], trows_v, sem).wait()
        pltpu.sync_copy(trows_v, out.at[6 + comp, pl.ds(base, _BPW)])

    for pair in range(6):
        tab = ctx_tabs[pair // 2]
        pltpu.sync_copy(ctx_idx.at[pair, wid], idx_v)

        def chunk_body(ch, carry, tab=tab):
            pltpu.async_copy(tab.at[idx_v.at[ch]], rows_v, sem).wait()
            for s in range(_SEGS_PER_CHUNK):
                accs = tuple(rows_v[s * _C, pl.ds(j * _L, _L)]
                             for j in range(_D // _L))

                def c_body(c, a, s=s):
                    return tuple(a[j] + rows_v[s * _C + c, pl.ds(j * _L, _L)]
                                 for j in range(_D // _L))

                accs = lax.fori_loop(1, _C, c_body, accs)
                seg = ch * _SEGS_PER_CHUNK + s
                for j in range(_D // _L):
                    acc_v[seg, pl.ds(j * _L, _L)] = accs[j]
            return carry

        lax.fori_loop(0, _CHUNKS, chunk_body, 0)
        pltpu.sync_copy(acc_v, out.at[pair, pl.ds(base, _BPW)])


_sc_gather_sums = functools.partial(
    pl.kernel,
    out_type=jax.ShapeDtypeStruct((9, _B, _D), jnp.float32),
    mesh=plsc.VectorSubcoreMesh(core_axis_name="c", subcore_axis_name="s"),
    scratch_types=[
        pltpu.VMEM((_CHUNKS, _ROWS), jnp.int32),
        pltpu.VMEM((_ROWS, _D), jnp.float32),
        pltpu.VMEM((_BPW, _D), jnp.float32),
        pltpu.VMEM((_BPW,), jnp.int32),
        pltpu.VMEM((_BPW, _D), jnp.float32),
        pltpu.SemaphoreType.DMA,
    ],
    compiler_params=pltpu.CompilerParams(use_tc_tiling_on_sc=False),
)(_sc_body)


def _loss_body(s_ref, o_ref):
    total = jnp.float32(0.0)
    for comp in range(3):
        t = s_ref[6 + comp]
        p = s_ref[2 * comp]
        n = s_ref[2 * comp + 1]
        pos = t * p + _EPS
        neg = (t * n + _EPS) - 1.0
        sp = jnp.maximum(-pos, 0.0) + jnp.log1p(jnp.exp(-jnp.abs(pos)))
        sn = jnp.maximum(neg, 0.0) + jnp.log1p(jnp.exp(-jnp.abs(neg)))
        total = total + (jnp.sum(sp) + jnp.sum(sn))
    o_ref[0, 0] = total / (_B * _D)


_loss_tc = pl.pallas_call(
    _loss_body,
    out_shape=jax.ShapeDtypeStruct((1, 1), jnp.float32),
    out_specs=pl.BlockSpec(memory_space=pltpu.SMEM),
)


def kernel(target_triples, pos_context, neg_context,
           W_target_head, W_target_tail, W_target_rel,
           W_context_head, W_context_tail, W_context_rel):
    tt = target_triples.astype(jnp.int32)
    pc = pos_context.astype(jnp.int32)
    ng = neg_context.astype(jnp.int32)

    tgt_idx = tt.T.reshape(3, _NW, _BPW)
    ctx_idx = jnp.stack([
        pc[:, :, 0], ng[:, :, 0],
        pc[:, :, 1], ng[:, :, 1],
        pc[:, :, 2], ng[:, :, 2],
    ]).reshape(6, _NW, _CHUNKS, _ROWS)

    sums = _sc_gather_sums(
        W_target_head, W_target_rel, W_target_tail,
        W_context_head, W_context_rel, W_context_tail,
        ctx_idx, tgt_idx)
    return _loss_tc(sums)[0, 0]
